# Initial kernel scaffold; baseline (speedup 1.0000x reference)
import jax
import jax.numpy as jnp
from jax.experimental import pallas as pl


def kernel(w, y, edge_index, pt):
    raise NotImplementedError("write your pallas kernel here")


# speedup vs baseline: 231.5396x; 231.5396x over previous
import functools

import jax
import jax.numpy as jnp
from jax import lax
from jax.experimental import pallas as pl
from jax.experimental.pallas import tpu as pltpu
from jax.experimental.pallas import tpu_sc as plsc

_NC = 2
_NS = 16
_NW = _NC * _NS
_L = 16

_PT_THLD = 0.9
_LN2 = 0.6931471805599453


def _ln(x):
    bits = plsc.bitcast(x, jnp.int32)
    e = (bits >> 23) - 127
    m = plsc.bitcast((bits & 0x007FFFFF) | 0x3F800000, jnp.float32)
    z = (m - 1.0) / (m + 1.0)
    z2 = z * z
    p = (1.0 / 3.0) + z2 * ((1.0 / 5.0) + z2 * (1.0 / 7.0))
    lnm = (2.0 * z) * (1.0 + z2 * p)
    return lnm + e.astype(jnp.float32) * _LN2


def _make_sc_loss(n_edges, n_nodes, chunk):
    epw = n_edges // _NW
    nchunks = epw // chunk
    vecs = chunk // _L
    mesh = plsc.VectorSubcoreMesh(core_axis_name="c", subcore_axis_name="s")

    @functools.partial(
        pl.kernel,
        out_type=jax.ShapeDtypeStruct((_NW, _L), jnp.float32),
        mesh=mesh,
        compiler_params=pltpu.CompilerParams(needs_layout_passes=False),
        scratch_types=[
            pltpu.VMEM((n_nodes,), jnp.float32),
            pltpu.VMEM((chunk,), jnp.float32),
            pltpu.VMEM((chunk,), jnp.int32),
            pltpu.VMEM((chunk,), jnp.int32),
            pltpu.VMEM((_L,), jnp.float32),
        ],
    )
    def sc_loss(w_hbm, y_hbm, ei_hbm, pt_hbm, out_hbm,
                pt_v, w_v, y_v, idx_v, acc_v):
        wid = lax.axis_index("s") * _NC + lax.axis_index("c")
        base = wid * epw
        pltpu.sync_copy(pt_hbm, pt_v)

        def chunk_body(c, acc):
            cbase = pl.multiple_of(base + c * chunk, 8)
            pltpu.sync_copy(w_hbm.at[pl.ds(cbase, chunk)], w_v)
            pltpu.sync_copy(y_hbm.at[pl.ds(cbase, chunk)], y_v)
            pltpu.sync_copy(ei_hbm.at[pl.ds(cbase, chunk)], idx_v)

            def vec_body(i, acc):
                sl = pl.ds(i * _L, _L)
                idx = idx_v[sl]
                g = plsc.load_gather(pt_v, [idx])
                wv = w_v[sl]
                yv = y_v[sl]
                ym = (yv != 0) & (g > _PT_THLD)
                sel = jnp.where(ym, wv, 1.0 - wv)
                loss = jnp.minimum(-_ln(sel), 100.0)
                return acc + loss

            return lax.fori_loop(0, vecs, vec_body, acc)

        acc = lax.fori_loop(0, nchunks, chunk_body,
                            jnp.zeros((_L,), jnp.float32))
        acc_v[...] = acc
        pltpu.sync_copy(acc_v, out_hbm.at[wid])

    return sc_loss


def kernel(w, y, edge_index, pt):
    n_edges = w.shape[0]
    n_nodes = pt.shape[0]
    sc_loss = _make_sc_loss(n_edges, n_nodes, chunk=4000)
    ei_flat = edge_index.astype(jnp.int32).reshape(-1)
    partials = sc_loss(w, y.astype(jnp.int32), ei_flat, pt)
    return jnp.sum(partials) / n_edges


# speedup vs baseline: 419.2400x; 1.8107x over previous
import functools

import jax
import jax.numpy as jnp
from jax import lax
from jax.experimental import pallas as pl
from jax.experimental.pallas import tpu as pltpu
from jax.experimental.pallas import tpu_sc as plsc

_NC = 2
_NS = 16
_NW = _NC * _NS
_L = 16

_PT_THLD = 0.9
_LN2 = 0.6931471805599453
_UNROLL = 2


def _ln(x):
    bits = plsc.bitcast(x, jnp.int32)
    e = (bits >> 23) - 127
    m = plsc.bitcast((bits & 0x007FFFFF) | 0x3F800000, jnp.float32)
    z = (m - 1.0) / (m + 1.0)
    z2 = z * z
    p = (1.0 / 3.0) + z2 * ((1.0 / 5.0) + z2 * (1.0 / 7.0))
    lnm = (2.0 * z) * (1.0 + z2 * p)
    return lnm + e.astype(jnp.float32) * _LN2


def _make_sc_loss(n_edges, n_nodes, chunk):
    epw = n_edges // _NW
    nchunks = epw // chunk
    vecs = chunk // _L
    assert epw * _NW == n_edges and nchunks * chunk == epw
    assert nchunks % 2 == 0 and vecs % _UNROLL == 0
    mesh = plsc.VectorSubcoreMesh(core_axis_name="c", subcore_axis_name="s")

    @functools.partial(
        pl.kernel,
        out_type=jax.ShapeDtypeStruct((_NW, _L), jnp.float32),
        mesh=mesh,
        compiler_params=pltpu.CompilerParams(needs_layout_passes=False),
        scratch_types=[
            pltpu.VMEM((n_nodes,), jnp.float32),
            pltpu.VMEM((chunk,), jnp.float32),
            pltpu.VMEM((chunk,), jnp.float32),
            pltpu.VMEM((chunk,), jnp.int32),
            pltpu.VMEM((chunk,), jnp.int32),
            pltpu.VMEM((chunk,), jnp.int32),
            pltpu.VMEM((chunk,), jnp.int32),
            pltpu.VMEM((_L,), jnp.float32),
            pltpu.SemaphoreType.DMA,
            pltpu.SemaphoreType.DMA,
        ],
    )
    def sc_loss(w_hbm, y_hbm, ei_hbm, pt_hbm, out_hbm,
                pt_v, w0_v, w1_v, y0_v, y1_v, idx0_v, idx1_v,
                acc_v, sem0, sem1):
        wid = lax.axis_index("s") * _NC + lax.axis_index("c")
        base = wid * epw
        sems = (sem0, sem1)
        w_v = (w0_v, w1_v)
        y_v = (y0_v, y1_v)
        idx_v = (idx0_v, idx1_v)
        pltpu.sync_copy(pt_hbm, pt_v)

        def copies(c, b):
            cbase = pl.multiple_of(base + c * chunk, 8)
            sl = pl.ds(cbase, chunk)
            return (
                pltpu.make_async_copy(w_hbm.at[sl], w_v[b], sems[b]),
                pltpu.make_async_copy(y_hbm.at[sl], y_v[b], sems[b]),
                pltpu.make_async_copy(ei_hbm.at[sl], idx_v[b], sems[b]),
            )

        def start(c, b):
            for cp in copies(c, b):
                cp.start()

        def wait(c, b):
            for cp in copies(c, b):
                cp.wait()

        def compute(b, acc):
            def vec_body(i, acc):
                o = i * (_L * _UNROLL)
                for k in range(_UNROLL):
                    sl = pl.ds(o + k * _L, _L)
                    idx = idx_v[b][sl]
                    g = plsc.load_gather(pt_v, [idx])
                    wv = w_v[b][sl]
                    yv = y_v[b][sl]
                    ym = (yv != 0) & (g > _PT_THLD)
                    sel = jnp.where(ym, wv, 1.0 - wv)
                    acc = acc + jnp.minimum(-_ln(sel), 100.0)
                return acc

            return lax.fori_loop(0, vecs // _UNROLL, vec_body, acc)

        start(0, 0)
        start(1, 1)

        def chunk_body(c2, acc):
            c0 = c2 * 2
            c1 = c0 + 1
            wait(c0, 0)
            acc = compute(0, acc)
            start((c0 + 2) % nchunks, 0)
            wait(c1, 1)
            acc = compute(1, acc)
            start((c1 + 2) % nchunks, 1)
            return acc

        acc = lax.fori_loop(0, nchunks // 2, chunk_body,
                            jnp.zeros((_L,), jnp.float32))
        wait(0, 0)
        wait(1, 1)
        acc_v[...] = acc
        pltpu.sync_copy(acc_v, out_hbm.at[wid])

    return sc_loss


def kernel(w, y, edge_index, pt):
    n_edges = w.shape[0]
    n_nodes = pt.shape[0]
    sc_loss = _make_sc_loss(n_edges, n_nodes, chunk=4000)
    ei_flat = edge_index.astype(jnp.int32).reshape(-1)
    partials = sc_loss(w, y.astype(jnp.int32), ei_flat, pt)
    return jnp.sum(partials) / n_edges


# speedup vs baseline: 421.5466x; 1.0055x over previous
import functools

import jax
import jax.numpy as jnp
from jax import lax
from jax.experimental import pallas as pl
from jax.experimental.pallas import tpu as pltpu
from jax.experimental.pallas import tpu_sc as plsc

_NC = 2
_NS = 16
_NW = _NC * _NS
_L = 16

_PT_THLD = 0.9
_LN2 = 0.6931471805599453
_UNROLL = 5


def _ln(x):
    bits = plsc.bitcast(x, jnp.int32)
    e = (bits >> 23) - 127
    m = plsc.bitcast((bits & 0x007FFFFF) | 0x3F800000, jnp.float32)
    z = (m - 1.0) / (m + 1.0)
    z2 = z * z
    p = (1.0 / 3.0) + z2 * ((1.0 / 5.0) + z2 * (1.0 / 7.0))
    lnm = (2.0 * z) * (1.0 + z2 * p)
    return lnm + e.astype(jnp.float32) * _LN2


def _make_sc_loss(n_edges, n_nodes, chunk):
    epw = n_edges // _NW
    nchunks = epw // chunk
    vecs = chunk // _L
    assert epw * _NW == n_edges and nchunks * chunk == epw
    assert nchunks % 2 == 0 and vecs % _UNROLL == 0
    mesh = plsc.VectorSubcoreMesh(core_axis_name="c", subcore_axis_name="s")

    @functools.partial(
        pl.kernel,
        out_type=jax.ShapeDtypeStruct((_NW, _L), jnp.float32),
        mesh=mesh,
        compiler_params=pltpu.CompilerParams(needs_layout_passes=False),
        scratch_types=[
            pltpu.VMEM((n_nodes,), jnp.float32),
            pltpu.VMEM((chunk,), jnp.float32),
            pltpu.VMEM((chunk,), jnp.float32),
            pltpu.VMEM((chunk,), jnp.int32),
            pltpu.VMEM((chunk,), jnp.int32),
            pltpu.VMEM((chunk,), jnp.int32),
            pltpu.VMEM((chunk,), jnp.int32),
            pltpu.VMEM((_L,), jnp.float32),
            pltpu.SemaphoreType.DMA,
            pltpu.SemaphoreType.DMA,
        ],
    )
    def sc_loss(w_hbm, y_hbm, ei_hbm, pt_hbm, out_hbm,
                pt_v, w0_v, w1_v, y0_v, y1_v, idx0_v, idx1_v,
                acc_v, sem0, sem1):
        wid = lax.axis_index("s") * _NC + lax.axis_index("c")
        base = wid * epw
        sems = (sem0, sem1)
        w_v = (w0_v, w1_v)
        y_v = (y0_v, y1_v)
        idx_v = (idx0_v, idx1_v)
        pltpu.sync_copy(pt_hbm, pt_v)

        def copies(c, b):
            cbase = pl.multiple_of(base + c * chunk, 8)
            sl = pl.ds(cbase, chunk)
            return (
                pltpu.make_async_copy(w_hbm.at[sl], w_v[b], sems[b]),
                pltpu.make_async_copy(y_hbm.at[sl], y_v[b], sems[b]),
                pltpu.make_async_copy(ei_hbm.at[sl], idx_v[b], sems[b]),
            )

        def start(c, b):
            for cp in copies(c, b):
                cp.start()

        def wait(c, b):
            for cp in copies(c, b):
                cp.wait()

        def compute(b, acc):
            def vec_body(i, acc):
                sl = pl.ds(i, _L)
                idx = idx_v[b][sl]
                g = plsc.load_gather(pt_v, [idx])
                wv = w_v[b][sl]
                yv = y_v[b][sl]
                ym = (yv != 0) & (g > _PT_THLD)
                sel = jnp.where(ym, wv, 1.0 - wv)
                return acc + jnp.minimum(-_ln(sel), 100.0)

            return plsc.parallel_loop(0, chunk, _L, unroll=_UNROLL,
                                      carry=acc)(vec_body)

        start(0, 0)
        start(1, 1)

        def chunk_body(c2, acc):
            c0 = c2 * 2
            c1 = c0 + 1
            wait(c0, 0)
            acc = compute(0, acc)
            start((c0 + 2) % nchunks, 0)
            wait(c1, 1)
            acc = compute(1, acc)
            start((c1 + 2) % nchunks, 1)
            return acc

        acc = lax.fori_loop(0, nchunks // 2, chunk_body,
                            jnp.zeros((_L,), jnp.float32))
        wait(0, 0)
        wait(1, 1)
        acc_v[...] = acc
        pltpu.sync_copy(acc_v, out_hbm.at[wid])

    return sc_loss


def kernel(w, y, edge_index, pt):
    n_edges = w.shape[0]
    n_nodes = pt.shape[0]
    sc_loss = _make_sc_loss(n_edges, n_nodes, chunk=4000)
    ei_flat = edge_index.astype(jnp.int32).reshape(-1)
    partials = sc_loss(w, y.astype(jnp.int32), ei_flat, pt)
    return jnp.sum(partials) / n_edges
